# Initial kernel scaffold; baseline (speedup 1.0000x reference)
import jax
import jax.numpy as jnp
from jax.experimental import pallas as pl


def kernel(x, edge_index, edge_label_index, is_directed, W_lin, b_lin, l1_W1, l1_b1, l1_g, l1_bt, l1_W2, l1_b2, l2_W1, l2_b1, l2_g, l2_bt, l2_W2, l2_b2, l3_W1, l3_b1, l3_g, l3_bt, l3_W2, l3_b2):
    raise NotImplementedError("write your pallas kernel here")


# speedup vs baseline: 2.8894x; 2.8894x over previous
import functools

import jax
import jax.numpy as jnp
from jax import lax
from jax.experimental import pallas as pl
from jax.experimental.pallas import tpu as pltpu
from jax.experimental.pallas import tpu_sc as plsc

N = 10000
E = 320000
EL = 100000
D = 128
H = 128

NC = 2
NS = 16
NW = NC * NS

K = 80
CH = 128
EPW = CH * K
EP = NW * EPW
NBUF = 4
NP = 10240
RPT = NP // NS
ZR = 32

ELP = 102400
PW = ELP // NW
CK = 128
CCH = PW // CK

_mesh = plsc.VectorSubcoreMesh(core_axis_name="c", subcore_axis_name="s")


def _seg_sum_body(h_hbm, src_hbm, dst_hbm, out_hbm,
                  spmem, srcb, dstb, rbuf, zbuf, gsem, ssem, dsem):
    c = lax.axis_index("c")
    s = lax.axis_index("s")
    w = c * NS + s

    @pl.loop(0, ZR)
    def _zfill(r):
        for j in range(D // 16):
            zbuf[r, pl.ds(j * 16, 16)] = jnp.zeros((16,), jnp.float32)

    for j in range(RPT // ZR):
        pltpu.sync_copy(zbuf, spmem.at[pl.ds(s * RPT + j * ZR, ZR)])

    plsc.subcore_barrier()

    for b in range(NBUF):
        pltpu.async_copy(src_hbm.at[w, b], srcb.at[b], ssem.at[b])
        pltpu.async_copy(dst_hbm.at[w, b], dstb.at[b], dsem.at[b])

    pltpu.make_async_copy(src_hbm.at[w, 0], srcb.at[0], ssem.at[0]).wait()
    pltpu.async_copy(h_hbm.at[srcb.at[0]], rbuf.at[0], gsem.at[0])

    @pl.loop(0, CH, step=NBUF)
    def _ring(g):
        for b in range(NBUF):
            i = g + b
            b1 = (b + 1) % NBUF
            pltpu.make_async_copy(
                h_hbm.at[srcb.at[b]], rbuf.at[b], gsem.at[b]).wait()

            @pl.when(i + 1 < CH)
            def _():
                pltpu.make_async_copy(
                    src_hbm.at[w, 0], srcb.at[b1], ssem.at[b1]).wait()
                pltpu.async_copy(
                    h_hbm.at[srcb.at[b1]], rbuf.at[b1], gsem.at[b1])

            pltpu.make_async_copy(
                dst_hbm.at[w, 0], dstb.at[b], dsem.at[b]).wait()
            pltpu.sync_copy(rbuf.at[b], spmem.at[dstb.at[b]], add=True)

            @pl.when(i + NBUF < CH)
            def _():
                pltpu.async_copy(
                    src_hbm.at[w, i + NBUF], srcb.at[b], ssem.at[b])
                pltpu.async_copy(
                    dst_hbm.at[w, i + NBUF], dstb.at[b], dsem.at[b])

    plsc.subcore_barrier()

    pltpu.sync_copy(spmem.at[pl.ds(s * RPT, RPT)],
                    out_hbm.at[c, pl.ds(s * RPT, RPT)])


def _seg_sum(h, src3d, dst3d):
    f = pl.kernel(
        _seg_sum_body,
        out_type=jax.ShapeDtypeStruct((NC, NP, D), jnp.float32),
        mesh=_mesh,
        scratch_types=[
            pltpu.VMEM_SHARED((NP, D), jnp.float32),
            pltpu.VMEM((NBUF, K), jnp.int32),
            pltpu.VMEM((NBUF, K), jnp.int32),
            pltpu.VMEM((NBUF, K, D), jnp.float32),
            pltpu.VMEM((ZR, D), jnp.float32),
            pltpu.SemaphoreType.DMA((NBUF,)),
            pltpu.SemaphoreType.DMA((NBUF,)),
            pltpu.SemaphoreType.DMA((NBUF,)),
        ],
    )
    return f(h, src3d, dst3d)


def _classifier_body(h_hbm, aidx_hbm, bidx_hbm, out_hbm,
                     aidx, bidx, abuf, bbuf, res2, sem):
    c = lax.axis_index("c")
    s = lax.axis_index("s")
    w = c * NS + s

    pltpu.sync_copy(aidx_hbm.at[w], aidx)
    pltpu.sync_copy(bidx_hbm.at[w], bidx)

    def start(i, p):
        pltpu.async_copy(h_hbm.at[aidx.at[i]], abuf.at[p], sem.at[2 * p])
        pltpu.async_copy(h_hbm.at[bidx.at[i]], bbuf.at[p], sem.at[2 * p + 1])

    def wait(i, p):
        pltpu.make_async_copy(
            h_hbm.at[aidx.at[i]], abuf.at[p], sem.at[2 * p]).wait()
        pltpu.make_async_copy(
            h_hbm.at[bidx.at[i]], bbuf.at[p], sem.at[2 * p + 1]).wait()

    def compute(i, p):
        @pl.loop(0, CK)
        def _rows(r):
            a0 = abuf.at[p]
            b0 = bbuf.at[p]
            acc = a0[r, pl.ds(0, 16)] * b0[r, pl.ds(0, 16)]
            for j in range(1, D // 16):
                acc = acc + a0[r, pl.ds(j * 16, 16)] * b0[r, pl.ds(j * 16, 16)]
            res2[r >> 3, pl.ds((r & 7) * 16, 16)] = acc

        pltpu.sync_copy(
            res2, out_hbm.at[pl.ds(w * (PW // 8) + i * (CK // 8), CK // 8)])

    start(0, 0)

    @pl.loop(0, CCH - 1, step=2)
    def _chunks(g):
        for p in range(2):
            i = g + p
            wait(i, p)
            start(i + 1, 1 - p)
            compute(i, p)

    wait(CCH - 1, 0)
    compute(CCH - 1, 0)


def _classifier(h, aidx3d, bidx3d):
    f = pl.kernel(
        _classifier_body,
        out_type=jax.ShapeDtypeStruct((ELP // 8, D), jnp.float32),
        mesh=_mesh,
        scratch_types=[
            pltpu.VMEM((CCH, CK), jnp.int32),
            pltpu.VMEM((CCH, CK), jnp.int32),
            pltpu.VMEM((2, CK, D), jnp.float32),
            pltpu.VMEM((2, CK, D), jnp.float32),
            pltpu.VMEM((CK // 8, D), jnp.float32),
            pltpu.SemaphoreType.DMA((4,)),
        ],
    )
    return f(h, aidx3d, bidx3d)


def _proj_body(x_ref, w_ref, b_ref, out_ref):
    out_ref[...] = jnp.dot(x_ref[...], w_ref[...],
                           preferred_element_type=jnp.float32) + b_ref[...]


def _proj(x, W, b):
    return pl.pallas_call(
        _proj_body,
        out_shape=jax.ShapeDtypeStruct((N, H), jnp.float32),
    )(x, W, b.reshape(1, H))


def _mlp_body(h_ref, a0_ref, a1_ref, w1_ref, b1_ref, g_ref, bt_ref,
              w2_ref, b2_ref, out_ref):
    z = h_ref[...] + a0_ref[pl.ds(0, N), :] + a1_ref[pl.ds(0, N), :]
    z1 = jnp.dot(z, w1_ref[...], preferred_element_type=jnp.float32) \
        + b1_ref[...]
    mu = jnp.mean(z1, axis=0, keepdims=True)
    z1c = z1 - mu
    var = jnp.mean(z1c * z1c, axis=0, keepdims=True)
    zn = z1c * lax.rsqrt(var + 1e-5) * g_ref[...] + bt_ref[...]
    zr = jnp.maximum(zn, 0.0)
    z2 = jnp.dot(zr, w2_ref[...], preferred_element_type=jnp.float32) \
        + b2_ref[...]
    out_ref[...] = jnp.maximum(z2, 0.0)


def _mlp(h, agg, W1, b1, g, bt, W2, b2):
    return pl.pallas_call(
        _mlp_body,
        out_shape=jax.ShapeDtypeStruct((N, H), jnp.float32),
    )(h, agg[0], agg[1], W1, b1.reshape(1, H), g.reshape(1, H),
      bt.reshape(1, H), W2, b2.reshape(1, H))


def _ereduce_body(p_ref, b_ref, out_ref):
    out_ref[...] = jnp.dot(p_ref[...], b_ref[...],
                           preferred_element_type=jnp.float32)


def _ereduce(p16, B):
    return pl.pallas_call(
        _ereduce_body,
        out_shape=jax.ShapeDtypeStruct((ELP // 8, 8), jnp.float32),
    )(p16, B)


def kernel(x, edge_index, edge_label_index, is_directed, W_lin, b_lin,
           l1_W1, l1_b1, l1_g, l1_bt, l1_W2, l1_b2,
           l2_W1, l2_b1, l2_g, l2_bt, l2_W2, l2_b2,
           l3_W1, l3_b1, l3_g, l3_bt, l3_W2, l3_b2):
    pad = EP - E
    src3d = jnp.concatenate(
        [edge_index[0], jnp.zeros((pad,), jnp.int32)]).reshape(NW, CH, K)
    dst3d = jnp.concatenate(
        [edge_index[1], jnp.full((pad,), NP - 1, jnp.int32)]).reshape(NW, CH, K)
    eli = jnp.pad(edge_label_index, ((0, 0), (0, ELP - EL)))
    aidx3d = eli[0].reshape(NW, CCH, CK)
    bidx3d = eli[1].reshape(NW, CCH, CK)
    B = jnp.equal(jnp.arange(D, dtype=jnp.int32)[:, None] // 16,
                  jnp.arange(8, dtype=jnp.int32)[None, :]).astype(jnp.float32)

    h = _proj(x, W_lin, b_lin)
    for (W1, b1, g, bt, W2, b2) in (
            (l1_W1, l1_b1, l1_g, l1_bt, l1_W2, l1_b2),
            (l2_W1, l2_b1, l2_g, l2_bt, l2_W2, l2_b2),
            (l3_W1, l3_b1, l3_g, l3_bt, l3_W2, l3_b2)):
        agg = _seg_sum(h, src3d, dst3d)
        h = _mlp(h, agg, W1, b1, g, bt, W2, b2)

    p16 = _classifier(h, aidx3d, bidx3d)
    pred = _ereduce(p16, B).reshape(ELP)
    return pred[:EL]


# speedup vs baseline: 4.6760x; 1.6183x over previous
import functools

import jax
import jax.numpy as jnp
from jax import lax
from jax.experimental import pallas as pl
from jax.experimental.pallas import tpu as pltpu
from jax.experimental.pallas import tpu_sc as plsc

N = 10000
E = 320000
EL = 100000
D = 128
H = 128

NC = 2
NS = 16
NW = NC * NS

K = 80
CH0 = 184
CH1 = 68
CHT = CH0 + CH1
EP = NS * CHT * K
NBUF = 4
NP = 10240
RPT = NP // NS
ZR = 32

CK = 128
CT = 50
X0 = 40
X1 = CT - X0
ELP = NS * CT * CK

_mesh = plsc.VectorSubcoreMesh(core_axis_name="c", subcore_axis_name="s")


def _seg_sum_body(h_hbm, src_hbm, dst_hbm, out_hbm,
                  spmem, srcb, dstb, rbuf, zbuf, gsem, ssem, dsem):
    c = lax.axis_index("c")
    s = lax.axis_index("s")
    base = c * CH0
    nch = CH0 - c * (CH0 - CH1)

    @pl.loop(0, ZR)
    def _zfill(r):
        for j in range(D // 16):
            zbuf[r, pl.ds(j * 16, 16)] = jnp.zeros((16,), jnp.float32)

    for j in range(RPT // ZR):
        pltpu.sync_copy(zbuf, spmem.at[pl.ds(s * RPT + j * ZR, ZR)])

    plsc.subcore_barrier()

    for b in range(NBUF):
        pltpu.async_copy(src_hbm.at[s, base + b], srcb.at[b], ssem.at[b])
        pltpu.async_copy(dst_hbm.at[s, base + b], dstb.at[b], dsem.at[b])

    pltpu.make_async_copy(src_hbm.at[s, base], srcb.at[0], ssem.at[0]).wait()
    pltpu.async_copy(h_hbm.at[srcb.at[0]], rbuf.at[0], gsem.at[0])

    @pl.loop(0, CH0, step=NBUF)
    def _ring(g):
        @pl.when(g < nch)
        def _():
            for b in range(NBUF):
                i = g + b
                b1 = (b + 1) % NBUF
                pltpu.make_async_copy(
                    h_hbm.at[srcb.at[b]], rbuf.at[b], gsem.at[b]).wait()

                @pl.when(i + 1 < nch)
                def _():
                    pltpu.make_async_copy(
                        src_hbm.at[s, base], srcb.at[b1], ssem.at[b1]).wait()
                    pltpu.async_copy(
                        h_hbm.at[srcb.at[b1]], rbuf.at[b1], gsem.at[b1])

                pltpu.make_async_copy(
                    dst_hbm.at[s, base], dstb.at[b], dsem.at[b]).wait()
                pltpu.sync_copy(rbuf.at[b], spmem.at[dstb.at[b]], add=True)

                @pl.when(i + NBUF < nch)
                def _():
                    pltpu.async_copy(
                        src_hbm.at[s, base + i + NBUF], srcb.at[b], ssem.at[b])
                    pltpu.async_copy(
                        dst_hbm.at[s, base + i + NBUF], dstb.at[b], dsem.at[b])

    plsc.subcore_barrier()

    pltpu.sync_copy(spmem.at[pl.ds(s * RPT, RPT)],
                    out_hbm.at[c, pl.ds(s * RPT, RPT)])


def _seg_sum(h, src3d, dst3d):
    f = pl.kernel(
        _seg_sum_body,
        out_type=jax.ShapeDtypeStruct((NC, NP, D), jnp.float32),
        mesh=_mesh,
        scratch_types=[
            pltpu.VMEM_SHARED((NP, D), jnp.float32),
            pltpu.VMEM((NBUF, K), jnp.int32),
            pltpu.VMEM((NBUF, K), jnp.int32),
            pltpu.VMEM((NBUF, K, D), jnp.float32),
            pltpu.VMEM((ZR, D), jnp.float32),
            pltpu.SemaphoreType.DMA((NBUF,)),
            pltpu.SemaphoreType.DMA((NBUF,)),
            pltpu.SemaphoreType.DMA((NBUF,)),
        ],
    )
    return f(h, src3d, dst3d)


def _classifier_body(h_hbm, aidx_hbm, bidx_hbm, out_hbm,
                     aidx, bidx, abuf, bbuf, res2, sem):
    c = lax.axis_index("c")
    s = lax.axis_index("s")
    base = c * X0
    nch = X0 - c * (X0 - X1)

    pltpu.sync_copy(aidx_hbm.at[s], aidx)
    pltpu.sync_copy(bidx_hbm.at[s], bidx)

    def start(i, p):
        pltpu.async_copy(h_hbm.at[aidx.at[base + i]], abuf.at[p],
                         sem.at[2 * p])
        pltpu.async_copy(h_hbm.at[bidx.at[base + i]], bbuf.at[p],
                         sem.at[2 * p + 1])

    def wait(i, p):
        pltpu.make_async_copy(
            h_hbm.at[aidx.at[base + i]], abuf.at[p], sem.at[2 * p]).wait()
        pltpu.make_async_copy(
            h_hbm.at[bidx.at[base + i]], bbuf.at[p], sem.at[2 * p + 1]).wait()

    def compute(i, p):
        @pl.loop(0, CK)
        def _rows(r):
            a0 = abuf.at[p]
            b0 = bbuf.at[p]
            acc = a0[r, pl.ds(0, 16)] * b0[r, pl.ds(0, 16)]
            for j in range(1, D // 16):
                acc = acc + a0[r, pl.ds(j * 16, 16)] * b0[r, pl.ds(j * 16, 16)]
            res2[r >> 3, pl.ds((r & 7) * 16, 16)] = acc

        pltpu.sync_copy(
            res2,
            out_hbm.at[pl.ds((s * CT + base + i) * (CK // 8), CK // 8)])

    start(0, 0)

    @pl.loop(0, X0, step=2)
    def _chunks(g):
        @pl.when(g < nch)
        def _():
            for p in range(2):
                i = g + p
                wait(i, p)

                @pl.when(i + 1 < nch)
                def _():
                    start(i + 1, 1 - p)

                compute(i, p)


def _classifier(h, aidx3d, bidx3d):
    f = pl.kernel(
        _classifier_body,
        out_type=jax.ShapeDtypeStruct((ELP // 8, D), jnp.float32),
        mesh=_mesh,
        scratch_types=[
            pltpu.VMEM((CT, CK), jnp.int32),
            pltpu.VMEM((CT, CK), jnp.int32),
            pltpu.VMEM((2, CK, D), jnp.float32),
            pltpu.VMEM((2, CK, D), jnp.float32),
            pltpu.VMEM((CK // 8, D), jnp.float32),
            pltpu.SemaphoreType.DMA((4,)),
        ],
    )
    return f(h, aidx3d, bidx3d)


def _proj_body(x_ref, w_ref, b_ref, out_ref):
    out_ref[...] = jnp.dot(x_ref[...], w_ref[...],
                           preferred_element_type=jnp.float32) + b_ref[...]


def _proj(x, W, b):
    return pl.pallas_call(
        _proj_body,
        out_shape=jax.ShapeDtypeStruct((N, H), jnp.float32),
    )(x, W, b.reshape(1, H))


def _mlp_body(h_ref, a0_ref, a1_ref, w1_ref, b1_ref, g_ref, bt_ref,
              w2_ref, b2_ref, out_ref):
    z = h_ref[...] + a0_ref[pl.ds(0, N), :] + a1_ref[pl.ds(0, N), :]
    z1 = jnp.dot(z, w1_ref[...], preferred_element_type=jnp.float32) \
        + b1_ref[...]
    mu = jnp.mean(z1, axis=0, keepdims=True)
    z1c = z1 - mu
    var = jnp.mean(z1c * z1c, axis=0, keepdims=True)
    zn = z1c * lax.rsqrt(var + 1e-5) * g_ref[...] + bt_ref[...]
    zr = jnp.maximum(zn, 0.0)
    z2 = jnp.dot(zr, w2_ref[...], preferred_element_type=jnp.float32) \
        + b2_ref[...]
    out_ref[...] = jnp.maximum(z2, 0.0)


def _mlp(h, agg, W1, b1, g, bt, W2, b2):
    return pl.pallas_call(
        _mlp_body,
        out_shape=jax.ShapeDtypeStruct((N, H), jnp.float32),
    )(h, agg[0], agg[1], W1, b1.reshape(1, H), g.reshape(1, H),
      bt.reshape(1, H), W2, b2.reshape(1, H))


def _ereduce_body(p_ref, b_ref, out_ref):
    out_ref[...] = jnp.dot(p_ref[...], b_ref[...],
                           preferred_element_type=jnp.float32)


def _ereduce(p16, B):
    return pl.pallas_call(
        _ereduce_body,
        out_shape=jax.ShapeDtypeStruct((ELP // 8, 8), jnp.float32),
    )(p16, B)


def kernel(x, edge_index, edge_label_index, is_directed, W_lin, b_lin,
           l1_W1, l1_b1, l1_g, l1_bt, l1_W2, l1_b2,
           l2_W1, l2_b1, l2_g, l2_bt, l2_W2, l2_b2,
           l3_W1, l3_b1, l3_g, l3_bt, l3_W2, l3_b2):
    pad = EP - E
    src3d = jnp.concatenate(
        [edge_index[0], jnp.zeros((pad,), jnp.int32)]).reshape(NS, CHT, K)
    dst3d = jnp.concatenate(
        [edge_index[1], jnp.full((pad,), NP - 1, jnp.int32)]).reshape(NS, CHT, K)
    eli = jnp.pad(edge_label_index, ((0, 0), (0, ELP - EL)))
    aidx3d = eli[0].reshape(NS, CT, CK)
    bidx3d = eli[1].reshape(NS, CT, CK)
    B = jnp.equal(jnp.arange(D, dtype=jnp.int32)[:, None] // 16,
                  jnp.arange(8, dtype=jnp.int32)[None, :]).astype(jnp.float32)

    h = _proj(x, W_lin, b_lin)
    for (W1, b1, g, bt, W2, b2) in (
            (l1_W1, l1_b1, l1_g, l1_bt, l1_W2, l1_b2),
            (l2_W1, l2_b1, l2_g, l2_bt, l2_W2, l2_b2),
            (l3_W1, l3_b1, l3_g, l3_bt, l3_W2, l3_b2)):
        agg = _seg_sum(h, src3d, dst3d)
        h = _mlp(h, agg, W1, b1, g, bt, W2, b2)

    p16 = _classifier(h, aidx3d, bidx3d)
    pred = _ereduce(p16, B).reshape(ELP)
    return pred[:EL]
